# baseline (speedup 1.0000x reference)
import jax
import jax.numpy as jnp
from jax import lax
from jax.experimental import pallas as pl
from jax.experimental.pallas import tpu as pltpu
from jax.experimental.pallas import tpu_sc as plsc

TOP_K = 2
NUM_EXPERTS = 8
DIM = 768
ALPHA = 0.001
BSZ = 4
SEQ = 8192

TOKENS = BSZ * SEQ
LANES = 16
NW = 32
TPW = TOKENS // NW
GROUPS = TPW // LANES
NEG_INF = float("-inf")


def _scores_body(x_ref, w_ref, s_ref):
    x = x_ref[...]
    w = w_ref[...]
    logits = lax.dot_general(
        w, x, (((1,), (1,)), ((), ())),
        preferred_element_type=jnp.float32)
    m = jnp.max(logits, axis=0, keepdims=True)
    ex = jnp.exp(logits - m)
    s_ref[...] = ex / jnp.sum(ex, axis=0, keepdims=True)


def _tc_scores(hs, weight):
    return pl.pallas_call(
        _scores_body,
        grid=(NW,),
        in_specs=[
            pl.BlockSpec((TPW, DIM), lambda i: (i, 0)),
            pl.BlockSpec((NUM_EXPERTS, DIM), lambda i: (0, 0)),
        ],
        out_specs=pl.BlockSpec((NUM_EXPERTS, TPW), lambda i: (i, 0)),
        out_shape=jax.ShapeDtypeStruct((NW * NUM_EXPERTS, TPW), jnp.float32),
    )(hs, weight)


def _routing_body(scores_hbm, idx_hbm, tw_hbm, pacc_hbm,
                  scores_v, idx_v, tw_v, pacc_v):
    nc = 2
    wid = lax.axis_index("s") * nc + lax.axis_index("c")
    base = wid * TPW

    pltpu.sync_copy(scores_hbm.at[pl.ds(wid * NUM_EXPERTS * TPW,
                                        NUM_EXPERTS * TPW)], scores_v)

    lane = lax.iota(jnp.int32, LANES)
    zf = jnp.zeros((LANES,), jnp.float32)

    def group(g, acc):
        cnt, ssum = acc
        off = g * LANES
        p = [scores_v[pl.ds(e * TPW + off, LANES)] for e in range(NUM_EXPERTS)]

        m1 = p[0]
        i1 = jnp.zeros((LANES,), jnp.int32)
        m2 = jnp.full((LANES,), NEG_INF, jnp.float32)
        i2 = jnp.zeros((LANES,), jnp.int32)
        for e in range(1, NUM_EXPERTS):
            pe = p[e]
            ei = jnp.full((LANES,), e, jnp.int32)
            gt1 = pe > m1
            gt2 = pe > m2
            i2 = jnp.where(gt1, i1, jnp.where(gt2, ei, i2))
            m2 = jnp.where(gt1, m1, jnp.where(gt2, pe, m2))
            i1 = jnp.where(gt1, ei, i1)
            m1 = jnp.where(gt1, pe, m1)

        r = 1.0 / (m1 + m2 + 1e-20)
        pos = (off + lane) * TOP_K
        plsc.store_scatter(idx_v, [pos], i1)
        plsc.store_scatter(idx_v, [pos + 1], i2)
        plsc.store_scatter(tw_v, [pos], m1 * r)
        plsc.store_scatter(tw_v, [pos + 1], m2 * r)

        cnt = [cnt[e]
               + jnp.where(i1 == e, 1.0, zf)
               + jnp.where(i2 == e, 1.0, zf)
               for e in range(NUM_EXPERTS)]
        ssum = [ssum[e] + p[e] for e in range(NUM_EXPERTS)]
        return (cnt, ssum)

    init = ([zf] * NUM_EXPERTS, [zf] * NUM_EXPERTS)
    cnt, ssum = lax.fori_loop(0, GROUPS, group, init)

    for e in range(NUM_EXPERTS):
        pacc_v[pl.ds(e * LANES, LANES)] = cnt[e]
        pacc_v[pl.ds((NUM_EXPERTS + e) * LANES, LANES)] = ssum[e]

    pltpu.sync_copy(idx_v, idx_hbm.at[pl.ds(base * TOP_K, TPW * TOP_K)])
    pltpu.sync_copy(tw_v, tw_hbm.at[pl.ds(base * TOP_K, TPW * TOP_K)])
    pltpu.sync_copy(pacc_v, pacc_hbm.at[pl.ds(wid * 2 * NUM_EXPERTS * LANES,
                                              2 * NUM_EXPERTS * LANES)])


_sc_routing = pl.kernel(
    _routing_body,
    out_type=(
        jax.ShapeDtypeStruct((TOKENS * TOP_K,), jnp.int32),
        jax.ShapeDtypeStruct((TOKENS * TOP_K,), jnp.float32),
        jax.ShapeDtypeStruct((NW * 2 * NUM_EXPERTS * LANES,), jnp.float32),
    ),
    mesh=plsc.VectorSubcoreMesh(core_axis_name="c", subcore_axis_name="s"),
    compiler_params=pltpu.CompilerParams(needs_layout_passes=False),
    scratch_types=[
        pltpu.VMEM((NUM_EXPERTS * TPW,), jnp.float32),
        pltpu.VMEM((TPW * TOP_K,), jnp.int32),
        pltpu.VMEM((TPW * TOP_K,), jnp.float32),
        pltpu.VMEM((2 * NUM_EXPERTS * LANES,), jnp.float32),
    ],
)


@jax.jit
def kernel(hidden_states, weight):
    hs = hidden_states.reshape(TOKENS, DIM)
    scores = _tc_scores(hs, weight).reshape(NW * NUM_EXPERTS * TPW)
    idx_flat, tw_flat, pacc = _sc_routing(scores)

    topk_idx = idx_flat.reshape(TOKENS, TOP_K)
    topk_w = tw_flat.reshape(TOKENS, TOP_K)

    pacc = pacc.reshape(NW, 2, NUM_EXPERTS, LANES).sum(-1)
    per_batch = pacc.reshape(BSZ, NW // BSZ, 2, NUM_EXPERTS).sum(1)
    cnt = per_batch[:, 0, :]
    ssum = per_batch[:, 1, :]
    ce = cnt * (NUM_EXPERTS / (SEQ * TOP_K))
    smean = ssum * (1.0 / SEQ)
    aux_loss = (ce * smean).sum(axis=1).mean() * ALPHA
    return (topk_idx, topk_w, aux_loss)


# speedup vs baseline: 1.0396x; 1.0396x over previous
import jax
import jax.numpy as jnp
from jax import lax
from jax.experimental import pallas as pl
from jax.experimental.pallas import tpu as pltpu
from jax.experimental.pallas import tpu_sc as plsc

TOP_K = 2
NUM_EXPERTS = 8
DIM = 768
ALPHA = 0.001
BSZ = 4
SEQ = 8192

TOKENS = BSZ * SEQ
LANES = 16
NW = 32
TPW = TOKENS // NW
GROUPS = TPW // LANES
NEG_INF = float("-inf")


def _scores_body(x_ref, w_ref, s_ref):
    x = x_ref[...]
    w = w_ref[...]
    logits = lax.dot_general(
        w, x, (((1,), (1,)), ((), ())),
        preferred_element_type=jnp.float32)
    m = jnp.max(logits, axis=0, keepdims=True)
    ex = jnp.exp(logits - m)
    s_ref[...] = ex / jnp.sum(ex, axis=0, keepdims=True)


BLOCK_T = 2048
GRID = TOKENS // BLOCK_T


def _tc_scores(hs, weight):
    return pl.pallas_call(
        _scores_body,
        grid=(GRID,),
        in_specs=[
            pl.BlockSpec((BLOCK_T, DIM), lambda i: (i, 0)),
            pl.BlockSpec((NUM_EXPERTS, DIM), lambda i: (0, 0)),
        ],
        out_specs=pl.BlockSpec((NUM_EXPERTS, BLOCK_T), lambda i: (0, i)),
        out_shape=jax.ShapeDtypeStruct((NUM_EXPERTS, TOKENS), jnp.float32),
    )(hs, weight)


def _routing_body(scores_hbm, idx_hbm, tw_hbm, pacc_hbm,
                  scores_v, idx_v, tw_v, pacc_v):
    nc = 2
    wid = lax.axis_index("s") * nc + lax.axis_index("c")
    base = wid * TPW

    for e in range(NUM_EXPERTS):
        pltpu.sync_copy(scores_hbm.at[pl.ds(e * TOKENS + base, TPW)],
                        scores_v.at[pl.ds(e * TPW, TPW)])

    lane = lax.iota(jnp.int32, LANES)
    zf = jnp.zeros((LANES,), jnp.float32)

    def group(g, acc):
        cnt, ssum = acc
        off = g * LANES
        p = [scores_v[pl.ds(e * TPW + off, LANES)] for e in range(NUM_EXPERTS)]

        m1 = p[0]
        i1 = jnp.zeros((LANES,), jnp.int32)
        m2 = jnp.full((LANES,), NEG_INF, jnp.float32)
        i2 = jnp.zeros((LANES,), jnp.int32)
        for e in range(1, NUM_EXPERTS):
            pe = p[e]
            ei = jnp.full((LANES,), e, jnp.int32)
            gt1 = pe > m1
            gt2 = pe > m2
            i2 = jnp.where(gt1, i1, jnp.where(gt2, ei, i2))
            m2 = jnp.where(gt1, m1, jnp.where(gt2, pe, m2))
            i1 = jnp.where(gt1, ei, i1)
            m1 = jnp.where(gt1, pe, m1)

        r = 1.0 / (m1 + m2 + 1e-20)
        pos = (off + lane) * TOP_K
        plsc.store_scatter(idx_v, [pos], i1)
        plsc.store_scatter(idx_v, [pos + 1], i2)
        plsc.store_scatter(tw_v, [pos], m1 * r)
        plsc.store_scatter(tw_v, [pos + 1], m2 * r)

        cnt = [cnt[e]
               + jnp.where(i1 == e, 1.0, zf)
               + jnp.where(i2 == e, 1.0, zf)
               for e in range(NUM_EXPERTS)]
        ssum = [ssum[e] + p[e] for e in range(NUM_EXPERTS)]
        return (cnt, ssum)

    init = ([zf] * NUM_EXPERTS, [zf] * NUM_EXPERTS)
    cnt, ssum = lax.fori_loop(0, GROUPS, group, init)

    for e in range(NUM_EXPERTS):
        pacc_v[pl.ds(e * LANES, LANES)] = cnt[e]
        pacc_v[pl.ds((NUM_EXPERTS + e) * LANES, LANES)] = ssum[e]

    pltpu.sync_copy(idx_v, idx_hbm.at[pl.ds(base * TOP_K, TPW * TOP_K)])
    pltpu.sync_copy(tw_v, tw_hbm.at[pl.ds(base * TOP_K, TPW * TOP_K)])
    pltpu.sync_copy(pacc_v, pacc_hbm.at[pl.ds(wid * 2 * NUM_EXPERTS * LANES,
                                              2 * NUM_EXPERTS * LANES)])


_sc_routing = pl.kernel(
    _routing_body,
    out_type=(
        jax.ShapeDtypeStruct((TOKENS * TOP_K,), jnp.int32),
        jax.ShapeDtypeStruct((TOKENS * TOP_K,), jnp.float32),
        jax.ShapeDtypeStruct((NW * 2 * NUM_EXPERTS * LANES,), jnp.float32),
    ),
    mesh=plsc.VectorSubcoreMesh(core_axis_name="c", subcore_axis_name="s"),
    compiler_params=pltpu.CompilerParams(needs_layout_passes=False),
    scratch_types=[
        pltpu.VMEM((NUM_EXPERTS * TPW,), jnp.float32),
        pltpu.VMEM((TPW * TOP_K,), jnp.int32),
        pltpu.VMEM((TPW * TOP_K,), jnp.float32),
        pltpu.VMEM((2 * NUM_EXPERTS * LANES,), jnp.float32),
    ],
)


@jax.jit
def kernel(hidden_states, weight):
    hs = hidden_states.reshape(TOKENS, DIM)
    scores = _tc_scores(hs, weight).reshape(NUM_EXPERTS * TOKENS)
    idx_flat, tw_flat, pacc = _sc_routing(scores)

    topk_idx = idx_flat.reshape(TOKENS, TOP_K)
    topk_w = tw_flat.reshape(TOKENS, TOP_K)

    pacc = pacc.reshape(NW, 2, NUM_EXPERTS, LANES).sum(-1)
    per_batch = pacc.reshape(BSZ, NW // BSZ, 2, NUM_EXPERTS).sum(1)
    cnt = per_batch[:, 0, :]
    ssum = per_batch[:, 1, :]
    ce = cnt * (NUM_EXPERTS / (SEQ * TOP_K))
    smean = ssum * (1.0 / SEQ)
    aux_loss = (ce * smean).sum(axis=1).mean() * ALPHA
    return (topk_idx, topk_w, aux_loss)
